# baseline (speedup 1.0000x reference)
import jax
import jax.numpy as jnp
from jax.experimental import pallas as pl
from jax.experimental.pallas import tpu as pltpu

CAP = 16384
B = 4096
ROW = 3 * 32 * 32

_WCHUNK = 512


def _winner_body(idx_col_ref, idx_row_ref, w_ref):
    col = idx_col_ref[...]
    row = idx_row_ref[...]
    match = col == row
    j = jax.lax.broadcasted_iota(jnp.int32, (_WCHUNK, B), 1)
    w_ref[...] = jnp.max(jnp.where(match, j, -1), axis=1, keepdims=True)


def _compute_winners(idx):
    return pl.pallas_call(
        _winner_body,
        grid=(B // _WCHUNK,),
        in_specs=[
            pl.BlockSpec((_WCHUNK, 1), lambda c: (c, 0)),
            pl.BlockSpec((1, B), lambda c: (0, 0)),
        ],
        out_specs=pl.BlockSpec((_WCHUNK, 1), lambda c: (c, 0)),
        out_shape=jax.ShapeDtypeStruct((B, 1), jnp.int32),
    )(idx.reshape(B, 1), idx.reshape(1, B)).reshape(B)


def _scatter_body(idx_ref, w_ref, val_ref, valy_ref, mem_ref, memy_ref,
                  out_ref, outy_ref):
    del idx_ref, w_ref, mem_ref, memy_ref
    out_ref[...] = val_ref[...]
    outy_ref[...] = valy_ref[...]


def kernel(mem_x, mem_y, idx, val_x, val_y):
    w = _compute_winners(idx)

    mem3 = mem_x.reshape(CAP, 1, ROW)
    val3 = val_x.reshape(B, 1, ROW)
    memy3 = mem_y.reshape(CAP, 1, 1)
    valy3 = val_y.reshape(B, 1, 1)

    grid_spec = pltpu.PrefetchScalarGridSpec(
        num_scalar_prefetch=2,
        grid=(B,),
        in_specs=[
            pl.BlockSpec((1, 1, ROW), lambda j, idx_ref, w_ref: (w_ref[j], 0, 0)),
            pl.BlockSpec((1, 1, 1), lambda j, idx_ref, w_ref: (w_ref[j], 0, 0)),
            pl.BlockSpec(memory_space=pl.ANY),
            pl.BlockSpec(memory_space=pl.ANY),
        ],
        out_specs=[
            pl.BlockSpec((1, 1, ROW), lambda j, idx_ref, w_ref: (idx_ref[j], 0, 0)),
            pl.BlockSpec((1, 1, 1), lambda j, idx_ref, w_ref: (idx_ref[j], 0, 0)),
        ],
    )
    out3, outy3 = pl.pallas_call(
        _scatter_body,
        grid_spec=grid_spec,
        out_shape=[
            jax.ShapeDtypeStruct((CAP, 1, ROW), jnp.float32),
            jax.ShapeDtypeStruct((CAP, 1, 1), jnp.int32),
        ],
        input_output_aliases={4: 0, 5: 1},
    )(idx, w, val3, valy3, mem3, memy3)

    return (out3.reshape(CAP, 3, 32, 32), outy3.reshape(CAP))


# speedup vs baseline: 5.3217x; 5.3217x over previous
import functools

import jax
import jax.numpy as jnp
from jax import lax
from jax.experimental import pallas as pl
from jax.experimental.pallas import tpu as pltpu
from jax.experimental.pallas import tpu_sc as plsc

CAP = 16384
B = 4096
ROW = 3 * 32 * 32

NC, NS, L = 2, 16, 16
NW = NC * NS
RPW = CAP // NW
JPW = B // NW
K = 16
NEG = -(2 ** 31) + 1

_WCHUNK = 512


def _winner_body(idx_col_ref, idx_row_ref, w_ref):
    col = idx_col_ref[...]
    row = idx_row_ref[...]
    match = col == row
    j = lax.broadcasted_iota(jnp.int32, (_WCHUNK, B), 1)
    w_ref[...] = jnp.max(jnp.where(match, j, -1), axis=1, keepdims=True)


def _compute_winners(idx):
    return pl.pallas_call(
        _winner_body,
        grid=(B // _WCHUNK,),
        in_specs=[
            pl.BlockSpec((_WCHUNK, 1), lambda c: (c, 0)),
            pl.BlockSpec((1, B), lambda c: (0, 0)),
        ],
        out_specs=pl.BlockSpec((_WCHUNK, 1), lambda c: (c, 0)),
        out_shape=jax.ShapeDtypeStruct((B, 1), jnp.int32),
    )(idx.reshape(B, 1), idx.reshape(1, B)).reshape(B)


def _sc_body(mem_hbm, val_hbm, idx_hbm, w_hbm, valy_hbm, memy_hbm, idx3_hbm,
             out_hbm, outy_hbm,
             idx_v, w_v, valy_v, myy_v, mark_v, kept1_v, kept2_v, tgt_v,
             buf0, buf1, sem0, sem1):
    wid = lax.axis_index("s") * NC + lax.axis_index("c")
    base = pl.multiple_of(wid * RPW, RPW)
    jbase = pl.multiple_of(wid * JPW, JPW)

    lanes = lax.broadcasted_iota(jnp.int32, (L,), 0)
    ones = jnp.ones((L,), jnp.int32)

    pltpu.sync_copy(idx_hbm, idx_v)
    pltpu.sync_copy(w_hbm, w_v)
    pltpu.sync_copy(valy_hbm, valy_v)
    pltpu.sync_copy(memy_hbm.at[pl.ds(base, RPW)], myy_v)
    pltpu.sync_copy(idx3_hbm.at[wid], tgt_v)

    for c in range(RPW // L):
        mark_v[pl.ds(c * L, L)] = jnp.zeros((L,), jnp.int32)

    def _scan_body(c, carry):
        off = pl.multiple_of(c * L, L)
        vi = idx_v[pl.ds(off, L)]
        vw = w_v[pl.ds(off, L)]
        data = plsc.load_gather(valy_v, [vw])
        rel = vi - base
        inrange = (rel >= 0) & (rel < RPW)
        relc = jnp.clip(rel, 0, RPW - 1)
        plsc.store_scatter(myy_v, [relc], data, mask=inrange)
        plsc.store_scatter(mark_v, [relc], ones, mask=inrange)
        return carry

    lax.fori_loop(0, B // L, _scan_body, jnp.int32(0))
    pltpu.sync_copy(myy_v, outy_hbm.at[pl.ds(base, RPW)])

    def _compact_body(c, nk):
        off = pl.multiple_of(c * L, L)
        flags = mark_v[pl.ds(off, L)]
        keep = flags == 0
        ids = base + off + lanes
        plsc.store_compressed(kept1_v.at[pl.ds(nk, L)], ids, mask=keep)
        cnt = plsc.all_reduce_population_count(keep)
        return nk + jnp.max(cnt, axis=0)

    nkeep = lax.fori_loop(0, RPW // L, _compact_body, jnp.int32(0))

    head = kept1_v[pl.ds(0, L)]
    first = jnp.max(jnp.where(lanes == 0, head, jnp.int32(NEG)), axis=0)

    @pl.when(nkeep > 0)
    def _():
        kept1_v[pl.ds(nkeep, L)] = jnp.full((L,), first, jnp.int32)

    for c in range(RPW // L):
        kept2_v[c, :] = kept1_v[pl.ds(c * L, L)]

    nchunks = (nkeep + (L - 1)) // L

    def _copy_body(c, carry):
        ilist = kept2_v.at[c]
        pltpu.async_copy(mem_hbm.at[ilist], buf0, sem0).wait()
        pltpu.async_copy(buf0, out_hbm.at[ilist], sem0).wait()
        return carry

    lax.fori_loop(0, nchunks, _copy_body, jnp.int32(0))

    for c in range(JPW // K):
        vw = w_v[pl.ds(pl.multiple_of(jbase + c * K, K), K)]
        pltpu.async_copy(val_hbm.at[vw], buf1, sem1).wait()
        pltpu.async_copy(buf1, out_hbm.at[tgt_v.at[c]], sem1).wait()


@functools.partial(jax.jit, static_argnames=())
def _sc_scatter(mem2, val2, idx, w, val_y, mem_y, idx3):
    mesh = plsc.VectorSubcoreMesh(core_axis_name="c", subcore_axis_name="s")
    f = pl.kernel(
        _sc_body,
        out_type=(
            jax.ShapeDtypeStruct((CAP, ROW), jnp.float32),
            jax.ShapeDtypeStruct((CAP,), jnp.int32),
        ),
        mesh=mesh,
        compiler_params=pltpu.CompilerParams(needs_layout_passes=False),
        scratch_types=[
            pltpu.VMEM((B,), jnp.int32),
            pltpu.VMEM((B,), jnp.int32),
            pltpu.VMEM((B,), jnp.int32),
            pltpu.VMEM((RPW,), jnp.int32),
            pltpu.VMEM((RPW,), jnp.int32),
            pltpu.VMEM((RPW + L,), jnp.int32),
            pltpu.VMEM((RPW // L, L), jnp.int32),
            pltpu.VMEM((JPW // K, K), jnp.int32),
            pltpu.VMEM((K, ROW), jnp.float32),
            pltpu.VMEM((K, ROW), jnp.float32),
            pltpu.SemaphoreType.DMA,
            pltpu.SemaphoreType.DMA,
        ],
    )
    return f(mem2, val2, idx, w, val_y, mem_y, idx3)


def kernel(mem_x, mem_y, idx, val_x, val_y):
    w = _compute_winners(idx)
    mem2 = mem_x.reshape(CAP, ROW)
    val2 = val_x.reshape(B, ROW)
    idx3 = idx.reshape(NW, JPW // K, K)
    out2, out_y = _sc_scatter(mem2, val2, idx, w, val_y, mem_y, idx3)
    return (out2.reshape(CAP, 3, 32, 32), out_y)


# speedup vs baseline: 5.5246x; 1.0381x over previous
import functools

import jax
import jax.numpy as jnp
from jax import lax
from jax.experimental import pallas as pl
from jax.experimental.pallas import tpu as pltpu
from jax.experimental.pallas import tpu_sc as plsc

CAP = 16384
B = 4096
ROW = 3 * 32 * 32

NC, NS, L = 2, 16, 16
NW = NC * NS
RPW = CAP // NW
JPW = B // NW
K = 16
NEG = -(2 ** 31) + 1

_WCHUNK = 512


def _winner_body(idx_col_ref, idx_row_ref, w_ref):
    col = idx_col_ref[...]
    row = idx_row_ref[...]
    match = col == row
    j = lax.broadcasted_iota(jnp.int32, (_WCHUNK, B), 1)
    w_ref[...] = jnp.max(jnp.where(match, j, -1), axis=1, keepdims=True)


def _compute_winners(idx):
    return pl.pallas_call(
        _winner_body,
        grid=(B // _WCHUNK,),
        in_specs=[
            pl.BlockSpec((_WCHUNK, 1), lambda c: (c, 0)),
            pl.BlockSpec((1, B), lambda c: (0, 0)),
        ],
        out_specs=pl.BlockSpec((_WCHUNK, 1), lambda c: (c, 0)),
        out_shape=jax.ShapeDtypeStruct((B, 1), jnp.int32),
    )(idx.reshape(B, 1), idx.reshape(1, B)).reshape(B)


def _sc_body(mem_hbm, val_hbm, idx_hbm, w_hbm, valy_hbm, memy_hbm, idx3_hbm,
             out_hbm, outy_hbm,
             idx_v, w_v, valy_v, myy_v, mark_v, kept1_v, kept2_v, tgt_v,
             buf0, buf1, semg0, semg1, sems0, sems1):
    wid = lax.axis_index("s") * NC + lax.axis_index("c")
    base = pl.multiple_of(wid * RPW, RPW)
    jbase = pl.multiple_of(wid * JPW, JPW)

    lanes = lax.broadcasted_iota(jnp.int32, (L,), 0)
    ones = jnp.ones((L,), jnp.int32)
    bufs = (buf0, buf1)
    semgs = (semg0, semg1)
    semss = (sems0, sems1)

    pltpu.sync_copy(idx_hbm, idx_v)
    pltpu.sync_copy(w_hbm, w_v)
    pltpu.sync_copy(valy_hbm, valy_v)
    pltpu.sync_copy(memy_hbm.at[pl.ds(base, RPW)], myy_v)
    pltpu.sync_copy(idx3_hbm.at[wid], tgt_v)

    for c in range(RPW // L):
        mark_v[pl.ds(c * L, L)] = jnp.zeros((L,), jnp.int32)

    def _scan_body(c, carry):
        off = pl.multiple_of(c * L, L)
        vi = idx_v[pl.ds(off, L)]
        vw = w_v[pl.ds(off, L)]
        data = plsc.load_gather(valy_v, [vw])
        rel = vi - base
        inrange = (rel >= 0) & (rel < RPW)
        relc = jnp.clip(rel, 0, RPW - 1)
        plsc.store_scatter(myy_v, [relc], data, mask=inrange)
        plsc.store_scatter(mark_v, [relc], ones, mask=inrange)
        return carry

    lax.fori_loop(0, B // L, _scan_body, jnp.int32(0))
    pltpu.sync_copy(myy_v, outy_hbm.at[pl.ds(base, RPW)])

    def _compact_body(c, nk):
        off = pl.multiple_of(c * L, L)
        flags = mark_v[pl.ds(off, L)]
        keep = flags == 0
        ids = base + off + lanes
        plsc.store_compressed(kept1_v.at[pl.ds(nk, L)], ids, mask=keep)
        cnt = plsc.all_reduce_population_count(keep)
        return nk + jnp.max(cnt, axis=0)

    nkeep = lax.fori_loop(0, RPW // L, _compact_body, jnp.int32(0))

    head = kept1_v[pl.ds(0, L)]
    first = jnp.max(jnp.where(lanes == 0, head, jnp.int32(NEG)), axis=0)

    @pl.when(nkeep > 0)
    def _():
        kept1_v[pl.ds(nkeep, L)] = jnp.full((L,), first, jnp.int32)

    for c in range(RPW // L):
        kept2_v[c, :] = kept1_v[pl.ds(c * L, L)]

    nchunks = (nkeep + (L - 1)) // L

    @pl.when(nchunks > 0)
    def _():
        pltpu.async_copy(mem_hbm.at[kept2_v.at[0]], buf0, semg0)

    def _copy_step(c, bx, gx, sx, by, gy, sy):
        @pl.when(c >= 1)
        def _():
            pltpu.make_async_copy(by, out_hbm.at[kept2_v.at[c - 1]], sy).wait()

        @pl.when(c + 1 < nchunks)
        def _():
            pltpu.async_copy(mem_hbm.at[kept2_v.at[c + 1]], by, gy)

        pltpu.make_async_copy(mem_hbm.at[kept2_v.at[c]], bx, gx).wait()
        pltpu.async_copy(bx, out_hbm.at[kept2_v.at[c]], sx)

    def _copy_body(c, carry):
        even = lax.rem(c, jnp.int32(2)) == 0

        @pl.when(even)
        def _():
            _copy_step(c, buf0, semg0, sems0, buf1, semg1, sems1)

        @pl.when(jnp.logical_not(even))
        def _():
            _copy_step(c, buf1, semg1, sems1, buf0, semg0, sems0)

        return carry

    lax.fori_loop(0, nchunks, _copy_body, jnp.int32(0))

    @pl.when(nchunks > 0)
    def _():
        lastc = nchunks - 1
        last_even = lax.rem(lastc, jnp.int32(2)) == 0

        @pl.when(last_even)
        def _():
            pltpu.make_async_copy(buf0, out_hbm.at[kept2_v.at[lastc]],
                                  sems0).wait()

        @pl.when(jnp.logical_not(last_even))
        def _():
            pltpu.make_async_copy(buf1, out_hbm.at[kept2_v.at[lastc]],
                                  sems1).wait()

    nsc = JPW // K
    vws = [w_v[pl.ds(pl.multiple_of(jbase + c * K, K), K)] for c in range(nsc)]
    pltpu.async_copy(val_hbm.at[vws[0]], buf0, semg0)
    for c in range(nsc):
        p = c % 2
        if c >= 1:
            pltpu.make_async_copy(bufs[1 - p], out_hbm.at[tgt_v.at[c - 1]],
                                  semss[1 - p]).wait()
        if c + 1 < nsc:
            pltpu.async_copy(val_hbm.at[vws[c + 1]], bufs[1 - p],
                             semgs[1 - p])
        pltpu.make_async_copy(val_hbm.at[vws[c]], bufs[p], semgs[p]).wait()
        pltpu.async_copy(bufs[p], out_hbm.at[tgt_v.at[c]], semss[p])
    pltpu.make_async_copy(bufs[(nsc - 1) % 2], out_hbm.at[tgt_v.at[nsc - 1]],
                          semss[(nsc - 1) % 2]).wait()


@functools.partial(jax.jit, static_argnames=())
def _sc_scatter(mem2, val2, idx, w, val_y, mem_y, idx3):
    mesh = plsc.VectorSubcoreMesh(core_axis_name="c", subcore_axis_name="s")
    f = pl.kernel(
        _sc_body,
        out_type=(
            jax.ShapeDtypeStruct((CAP, ROW), jnp.float32),
            jax.ShapeDtypeStruct((CAP,), jnp.int32),
        ),
        mesh=mesh,
        compiler_params=pltpu.CompilerParams(needs_layout_passes=False),
        scratch_types=[
            pltpu.VMEM((B,), jnp.int32),
            pltpu.VMEM((B,), jnp.int32),
            pltpu.VMEM((B,), jnp.int32),
            pltpu.VMEM((RPW,), jnp.int32),
            pltpu.VMEM((RPW,), jnp.int32),
            pltpu.VMEM((RPW + L,), jnp.int32),
            pltpu.VMEM((RPW // L, L), jnp.int32),
            pltpu.VMEM((JPW // K, K), jnp.int32),
            pltpu.VMEM((K, ROW), jnp.float32),
            pltpu.VMEM((K, ROW), jnp.float32),
            pltpu.SemaphoreType.DMA,
            pltpu.SemaphoreType.DMA,
            pltpu.SemaphoreType.DMA,
            pltpu.SemaphoreType.DMA,
        ],
    )
    return f(mem2, val2, idx, w, val_y, mem_y, idx3)


def kernel(mem_x, mem_y, idx, val_x, val_y):
    w = _compute_winners(idx)
    mem2 = mem_x.reshape(CAP, ROW)
    val2 = val_x.reshape(B, ROW)
    idx3 = idx.reshape(NW, JPW // K, K)
    out2, out_y = _sc_scatter(mem2, val2, idx, w, val_y, mem_y, idx3)
    return (out2.reshape(CAP, 3, 32, 32), out_y)


# speedup vs baseline: 6.8734x; 1.2441x over previous
import jax
import jax.numpy as jnp
from jax import lax
from jax.experimental import pallas as pl
from jax.experimental.pallas import tpu as pltpu
from jax.experimental.pallas import tpu_sc as plsc

CAP = 16384
B = 4096
ROW = 3 * 32 * 32

NC, NS, L = 2, 16, 16
NW = NC * NS
RPW = CAP // NW
JPW = B // NW
K = 16

_WCHUNK = 512


def _winner_body(idx_col_ref, idx_row_ref, w_ref):
    col = idx_col_ref[...]
    row = idx_row_ref[...]
    match = col == row
    j = lax.broadcasted_iota(jnp.int32, (_WCHUNK, B), 1)
    w_ref[...] = jnp.max(jnp.where(match, j, -1), axis=1, keepdims=True)


def _compute_winners(idx):
    return pl.pallas_call(
        _winner_body,
        grid=(B // _WCHUNK,),
        in_specs=[
            pl.BlockSpec((_WCHUNK, 1), lambda c: (c, 0)),
            pl.BlockSpec((1, B), lambda c: (0, 0)),
        ],
        out_specs=pl.BlockSpec((_WCHUNK, 1), lambda c: (c, 0)),
        out_shape=jax.ShapeDtypeStruct((B, 1), jnp.int32),
    )(idx.reshape(B, 1), idx.reshape(1, B)).reshape(B)


def _sc_body(val_hbm, idx_hbm, w_hbm, valy_hbm, idx3_hbm, x_hbm, y_hbm,
             idx_v, w_v, valy_v, myy_v, tgt_v,
             buf0, buf1, semg0, semg1, sems0, sems1):
    wid = lax.axis_index("s") * NC + lax.axis_index("c")
    base = pl.multiple_of(wid * RPW, RPW)
    jbase = pl.multiple_of(wid * JPW, JPW)

    bufs = (buf0, buf1)
    semgs = (semg0, semg1)
    semss = (sems0, sems1)

    pltpu.sync_copy(idx_hbm, idx_v)
    pltpu.sync_copy(w_hbm, w_v)
    pltpu.sync_copy(valy_hbm, valy_v)
    pltpu.sync_copy(y_hbm.at[pl.ds(base, RPW)], myy_v)
    pltpu.sync_copy(idx3_hbm.at[wid], tgt_v)

    nsc = JPW // K
    vws = [w_v[pl.ds(pl.multiple_of(jbase + c * K, K), K)] for c in range(nsc)]
    pltpu.async_copy(val_hbm.at[vws[0]], buf0, semg0)
    for c in range(nsc):
        p = c % 2
        if c >= 1:
            pltpu.make_async_copy(bufs[1 - p], x_hbm.at[tgt_v.at[c - 1]],
                                  semss[1 - p]).wait()
        if c + 1 < nsc:
            pltpu.async_copy(val_hbm.at[vws[c + 1]], bufs[1 - p],
                             semgs[1 - p])
        pltpu.make_async_copy(val_hbm.at[vws[c]], bufs[p], semgs[p]).wait()
        pltpu.async_copy(bufs[p], x_hbm.at[tgt_v.at[c]], semss[p])

    def _scan_body(c, carry):
        off = pl.multiple_of(c * L, L)
        vi = idx_v[pl.ds(off, L)]
        vw = w_v[pl.ds(off, L)]
        data = plsc.load_gather(valy_v, [vw])
        rel = vi - base
        inrange = (rel >= 0) & (rel < RPW)
        relc = jnp.clip(rel, 0, RPW - 1)
        plsc.store_scatter(myy_v, [relc], data, mask=inrange)
        return carry

    lax.fori_loop(0, B // L, _scan_body, jnp.int32(0))
    pltpu.sync_copy(myy_v, y_hbm.at[pl.ds(base, RPW)])

    pltpu.make_async_copy(bufs[(nsc - 1) % 2], x_hbm.at[tgt_v.at[nsc - 1]],
                          semss[(nsc - 1) % 2]).wait()


def _sc_scatter(x_ref, y_ref, val2, idx, w, val_y, idx3):
    mesh = plsc.VectorSubcoreMesh(core_axis_name="c", subcore_axis_name="s")
    f = pl.kernel(
        _sc_body,
        out_type=(),
        mesh=mesh,
        compiler_params=pltpu.CompilerParams(needs_layout_passes=False),
        scratch_types=[
            pltpu.VMEM((B,), jnp.int32),
            pltpu.VMEM((B,), jnp.int32),
            pltpu.VMEM((B,), jnp.int32),
            pltpu.VMEM((RPW,), jnp.int32),
            pltpu.VMEM((JPW // K, K), jnp.int32),
            pltpu.VMEM((K, ROW), jnp.float32),
            pltpu.VMEM((K, ROW), jnp.float32),
            pltpu.SemaphoreType.DMA,
            pltpu.SemaphoreType.DMA,
            pltpu.SemaphoreType.DMA,
            pltpu.SemaphoreType.DMA,
        ],
    )
    f(val2, idx, w, val_y, idx3, x_ref, y_ref)


def kernel(mem_x, mem_y, idx, val_x, val_y):
    w = _compute_winners(idx)
    mem2 = mem_x.reshape(CAP, ROW)
    val2 = val_x.reshape(B, ROW)
    idx3 = idx.reshape(NW, JPW // K, K)
    x_ref = jax.new_ref(mem2)
    y_ref = jax.new_ref(mem_y)
    _sc_scatter(x_ref, y_ref, val2, idx, w, val_y, idx3)
    return (x_ref[...].reshape(CAP, 3, 32, 32), y_ref[...])
